# baseline (speedup 1.0000x reference)
import functools

import jax
import jax.numpy as jnp
from jax.experimental import pallas as pl
from jax.experimental.pallas import tpu as pltpu

NUM_EXPERTS = 64
TOP_K = 8


def _logits_body(x_ref, w_ref, b_ref, out_ref):
    x = x_ref[...]
    w = w_ref[...]
    out_ref[...] = jax.lax.dot_general(
        x, w, (((1,), (1,)), ((), ())), preferred_element_type=jnp.float32
    ) + b_ref[...][None, :]


def _router_logits(flat_x, W, b):
    n_tokens, hidden = flat_x.shape
    blk = 1024
    grid = n_tokens // blk
    return pl.pallas_call(
        _logits_body,
        grid=(grid,),
        in_specs=[
            pl.BlockSpec((blk, hidden), lambda i: (i, 0)),
            pl.BlockSpec((NUM_EXPERTS, hidden), lambda i: (0, 0)),
            pl.BlockSpec((NUM_EXPERTS,), lambda i: (0,)),
        ],
        out_specs=pl.BlockSpec((blk, NUM_EXPERTS), lambda i: (i, 0)),
        out_shape=jax.ShapeDtypeStruct((n_tokens, NUM_EXPERTS), jnp.float32),
    )(flat_x, W, b)


def kernel(token_inputs, expert_capacity, W, b):
    num_groups, tokens_per_group, hidden_dim = token_inputs.shape
    num_experts = W.shape[0]
    flat_inputs = token_inputs.reshape(-1, hidden_dim)
    router_logits = _router_logits(flat_inputs, W, b)
    router_logits = router_logits.reshape(num_groups, tokens_per_group, num_experts)
    router_probs = jax.nn.softmax(router_logits, axis=-1)
    combine_weights, expert_indices = jax.lax.top_k(router_probs, TOP_K)
    expert_mask = jax.nn.one_hot(expert_indices, num_experts, dtype=jnp.float32)
    expert_mask = expert_mask.sum(axis=2)
    tokens_per_expert = expert_mask.sum(axis=(0, 1))
    total_tokens = tokens_per_expert.sum()
    target_tokens_per_expert = total_tokens / num_experts
    auxiliary_loss = jnp.mean((tokens_per_expert - target_tokens_per_expert) ** 2)
    router_z_loss = jnp.mean(router_logits ** 2)
    batch_size = num_groups * tokens_per_group
    batch_indices = jnp.repeat(jnp.arange(batch_size, dtype=jnp.int32), TOP_K)
    expert_indices_flat = expert_indices.reshape(-1).astype(jnp.int32)
    dispatch_indices = jnp.stack([batch_indices, expert_indices_flat], axis=1)
    dispatch_indices = dispatch_indices.reshape(num_groups, tokens_per_group, TOP_K, 2)
    return (dispatch_indices, combine_weights, auxiliary_loss, router_z_loss)


# speedup vs baseline: 1.3651x; 1.3651x over previous
import functools

import jax
import jax.numpy as jnp
from jax import lax
from jax.experimental import pallas as pl
from jax.experimental.pallas import tpu as pltpu
from jax.experimental.pallas import tpu_sc as plsc

NUM_EXPERTS = 64
TOP_K = 8
LANES = 16
NUM_TILES = 32
BLK = 1024


def _logits_body(x_ref, w_ref, b_ref, out_ref, z_ref):
    i = pl.program_id(0)
    lt = jax.lax.dot_general(
        w_ref[...], x_ref[...], (((1,), (1,)), ((), ())),
        preferred_element_type=jnp.float32,
    ) + b_ref[...][:, None]
    out_ref[0] = lt
    zpart = jnp.sum(lt * lt).reshape(1, 1)

    @pl.when(i == 0)
    def _():
        z_ref[...] = zpart

    @pl.when(i > 0)
    def _():
        z_ref[...] += zpart


def _router_logits_t(flat_x, W, b):
    n_tokens, hidden = flat_x.shape
    grid = n_tokens // BLK
    return pl.pallas_call(
        _logits_body,
        grid=(grid,),
        in_specs=[
            pl.BlockSpec((BLK, hidden), lambda i: (i, 0)),
            pl.BlockSpec((NUM_EXPERTS, hidden), lambda i: (0, 0)),
            pl.BlockSpec((NUM_EXPERTS,), lambda i: (0,)),
        ],
        out_specs=[
            pl.BlockSpec((1, NUM_EXPERTS, BLK), lambda i: (i, 0, 0)),
            pl.BlockSpec((1, 1), lambda i: (0, 0)),
        ],
        out_shape=[
            jax.ShapeDtypeStruct((grid, NUM_EXPERTS, BLK), jnp.float32),
            jax.ShapeDtypeStruct((1, 1), jnp.float32),
        ],
    )(flat_x, W, b)


def _routing_body(logits_hbm, cw_hbm, ei_hbm, hist_hbm, buf, cwf, eif, hist):
    info = plsc.get_sparse_core_info()
    nc = info.num_cores
    wid = lax.axis_index("s") * nc + lax.axis_index("c")

    pltpu.sync_copy(logits_hbm.at[wid], buf)

    lane = lax.iota(jnp.int32, 16)
    neg_inf = jnp.full((16,), -jnp.inf, jnp.float32)
    zero_i = jnp.zeros((16,), jnp.int32)
    one_i = jnp.ones((16,), jnp.int32)

    def init_hist(i, c):
        hist[pl.ds(i * 16, 16)] = zero_i
        return c

    lax.fori_loop(0, NUM_EXPERTS, init_hist, 0, unroll=8)

    def group_body(g, carry):
        col = g * 16
        colv = col + lane

        def max_body(e, m):
            return jnp.maximum(m, buf[e, pl.ds(col, 16)])

        m = lax.fori_loop(0, NUM_EXPERTS, max_body, neg_inf, unroll=8)

        def exp_body(e, s):
            return s + jnp.exp(buf[e, pl.ds(col, 16)] - m)

        s = lax.fori_loop(0, NUM_EXPERTS, exp_body,
                          jnp.zeros((16,), jnp.float32), unroll=8)
        rcp = 1.0 / s

        def k_body(k, carry2):
            def am_body(e, mi):
                m2, idx = mi
                v = buf[e, pl.ds(col, 16)]
                pred = v > m2
                return jnp.maximum(m2, v), jnp.where(pred, e, idx)

            m2, idx = lax.fori_loop(0, NUM_EXPERTS, am_body,
                                    (neg_inf, zero_i), unroll=8)
            w = jnp.exp(m2 - m) * rcp
            flat = colv * TOP_K + k
            plsc.store_scatter(cwf, [flat], w)
            plsc.store_scatter(eif, [flat], idx)
            plsc.addupdate_scatter(hist, [idx * 16 + lane], one_i)
            plsc.store_scatter(buf, [idx, colv], neg_inf)
            return carry2

        lax.fori_loop(0, TOP_K, k_body, 0)
        return carry

    lax.fori_loop(0, BLK // 16, group_body, 0)

    pltpu.sync_copy(cwf, cw_hbm.at[wid])
    pltpu.sync_copy(eif, ei_hbm.at[wid])
    pltpu.sync_copy(hist, hist_hbm.at[wid])


def _routing(logits_t):
    mesh = plsc.VectorSubcoreMesh(core_axis_name="c", subcore_axis_name="s")
    return pl.kernel(
        _routing_body,
        out_type=[
            jax.ShapeDtypeStruct((NUM_TILES, BLK * TOP_K), jnp.float32),
            jax.ShapeDtypeStruct((NUM_TILES, BLK * TOP_K), jnp.int32),
            jax.ShapeDtypeStruct((NUM_TILES, NUM_EXPERTS * LANES), jnp.int32),
        ],
        mesh=mesh,
        compiler_params=pltpu.CompilerParams(needs_layout_passes=False),
        scratch_types=[
            pltpu.VMEM((NUM_EXPERTS, BLK), jnp.float32),
            pltpu.VMEM((BLK * TOP_K,), jnp.float32),
            pltpu.VMEM((BLK * TOP_K,), jnp.int32),
            pltpu.VMEM((NUM_EXPERTS * LANES,), jnp.int32),
        ],
    )(logits_t)


def kernel(token_inputs, expert_capacity, W, b):
    num_groups, tokens_per_group, hidden_dim = token_inputs.shape
    num_experts = W.shape[0]
    batch_size = num_groups * tokens_per_group

    flat_inputs = token_inputs.reshape(-1, hidden_dim)
    logits_t, z_sum = _router_logits_t(flat_inputs, W, b)
    cw, ei, hist = _routing(logits_t)

    combine_weights = cw.reshape(num_groups, tokens_per_group, TOP_K)
    expert_indices = ei.reshape(batch_size * TOP_K)

    tokens_per_expert = hist.reshape(NUM_TILES, num_experts, LANES).astype(
        jnp.float32).sum(axis=(0, 2))
    target = tokens_per_expert.sum() / num_experts
    auxiliary_loss = jnp.mean((tokens_per_expert - target) ** 2)

    router_z_loss = z_sum[0, 0] / (batch_size * num_experts)

    batch_indices = jnp.repeat(
        jnp.arange(batch_size, dtype=jnp.int32), TOP_K)
    dispatch_indices = jnp.stack([batch_indices, expert_indices], axis=1)
    dispatch_indices = dispatch_indices.reshape(
        num_groups, tokens_per_group, TOP_K, 2)
    return (dispatch_indices, combine_weights, auxiliary_loss, router_z_loss)
